# baseline (speedup 1.0000x reference)
import functools
import math

import jax
import jax.numpy as jnp
from jax.experimental import pallas as pl
from jax.experimental.pallas import tpu as pltpu

_SMOOTHING = 0.1
_CONFIDENCE = 1.0 - _SMOOTHING


def _loss_kernel(inp_ref, tgt_ref, mask_ref, out_ref, asum_ref, atgt_ref, *,
                 nblocks, vb, V, C):
    b = pl.program_id(0)

    @pl.when(b == 0)
    def _init():
        asum_ref[:, :] = jnp.zeros_like(asum_ref)
        atgt_ref[:, :] = jnp.zeros_like(atgt_ref)

    x = inp_ref[:, :]
    col = jax.lax.broadcasted_iota(jnp.int32, x.shape, 1)
    hit = col == tgt_ref[:, :] - b * vb

    @pl.when(b < nblocks - 1)
    def _full():
        asum_ref[:, :] += jnp.sum(x, axis=1, keepdims=True)
        atgt_ref[:, :] += jnp.sum(jnp.where(hit, x, 0.0), axis=1, keepdims=True)

    @pl.when(b == nblocks - 1)
    def _tail():
        xz = jnp.where(col < V - b * vb, x, 0.0)
        asum = asum_ref[:, :] + jnp.sum(xz, axis=1, keepdims=True)
        atgt = atgt_ref[:, :] + jnp.sum(jnp.where(hit, xz, 0.0), axis=1,
                                        keepdims=True)
        eps = _SMOOTHING / (V - 1)
        m = mask_ref[:, :]
        kl = C - eps * asum - (_CONFIDENCE - eps) * atgt
        num = jnp.sum(m * kl, keepdims=True)
        den = jnp.sum(m, keepdims=True)
        out_ref[:, :] = num / den


def kernel(input, target, mask):
    S = input.shape[1]
    V = input.shape[-1]
    target = target[:, :S]
    mask = mask[:, :S]
    inp = input.reshape(-1, V)
    N = inp.shape[0]
    tgt = target.reshape(N, 1).astype(jnp.int32)
    m = mask.reshape(N, 1).astype(jnp.float32)

    eps = _SMOOTHING / (V - 1)
    C = (V - 1) * eps * math.log(eps) + _CONFIDENCE * math.log(_CONFIDENCE)

    VB = 8192
    nblocks = pl.cdiv(V, VB)

    out = pl.pallas_call(
        functools.partial(_loss_kernel, nblocks=nblocks, vb=VB, V=V, C=C),
        grid=(nblocks,),
        in_specs=[
            pl.BlockSpec((N, VB), lambda b: (0, b)),
            pl.BlockSpec((N, 1), lambda b: (0, 0)),
            pl.BlockSpec((N, 1), lambda b: (0, 0)),
        ],
        out_specs=pl.BlockSpec((1, 1), lambda b: (0, 0)),
        out_shape=jax.ShapeDtypeStruct((1, 1), jnp.float32),
        scratch_shapes=[pltpu.VMEM((N, 1), jnp.float32),
                        pltpu.VMEM((N, 1), jnp.float32)],
    )(inp, tgt, m)
    return out[0, 0]


# speedup vs baseline: 1.0302x; 1.0302x over previous
import functools
import math

import jax
import jax.numpy as jnp
from jax.experimental import pallas as pl
from jax.experimental.pallas import tpu as pltpu

_SMOOTHING = 0.1
_CONFIDENCE = 1.0 - _SMOOTHING


def _loss_kernel(inp_ref, tgt_ref, mask_ref, out_ref, num_ref, den_ref, *,
                 nblocks, V, C):
    b = pl.program_id(0)

    @pl.when(b == 0)
    def _init():
        num_ref[:, :] = jnp.zeros_like(num_ref)
        den_ref[:, :] = jnp.zeros_like(den_ref)

    eps = _SMOOTHING / (V - 1)
    x = inp_ref[:, :]
    col = jax.lax.broadcasted_iota(jnp.int32, x.shape, 1)
    x = jnp.where(col < V, x, 0.0)
    hit = col == tgt_ref[:, :]
    asum = jnp.sum(x, axis=1, keepdims=True)
    atgt = jnp.sum(jnp.where(hit, x, 0.0), axis=1, keepdims=True)
    m = mask_ref[:, :]
    kl = C - eps * asum - (_CONFIDENCE - eps) * atgt
    num_ref[:, :] += jnp.sum(m * kl, keepdims=True)
    den_ref[:, :] += jnp.sum(m, keepdims=True)

    @pl.when(b == nblocks - 1)
    def _finish():
        out_ref[:, :] = num_ref[:, :] / den_ref[:, :]


def kernel(input, target, mask):
    S = input.shape[1]
    V = input.shape[-1]
    target = target[:, :S]
    mask = mask[:, :S]
    inp = input.reshape(-1, V)
    N = inp.shape[0]
    tgt = target.reshape(N, 1).astype(jnp.int32)
    m = mask.reshape(N, 1).astype(jnp.float32)

    eps = _SMOOTHING / (V - 1)
    C = (V - 1) * eps * math.log(eps) + _CONFIDENCE * math.log(_CONFIDENCE)

    R = 32
    nblocks = N // R

    out = pl.pallas_call(
        functools.partial(_loss_kernel, nblocks=nblocks, V=V, C=C),
        grid=(nblocks,),
        in_specs=[
            pl.BlockSpec((R, V), lambda b: (b, 0)),
            pl.BlockSpec((R, 1), lambda b: (b, 0)),
            pl.BlockSpec((R, 1), lambda b: (b, 0)),
        ],
        out_specs=pl.BlockSpec((1, 1), lambda b: (0, 0)),
        out_shape=jax.ShapeDtypeStruct((1, 1), jnp.float32),
        scratch_shapes=[pltpu.VMEM((1, 1), jnp.float32),
                        pltpu.VMEM((1, 1), jnp.float32)],
    )(inp, tgt, m)
    return out[0, 0]
